# Initial kernel scaffold; baseline (speedup 1.0000x reference)
import jax
import jax.numpy as jnp
from jax.experimental import pallas as pl


def kernel(x, edge_index, edge_attr, batch, Wf1, bf1, Ws1, bs1, gamma1, beta1, Wf2, bf2, Ws2, bs2, gamma2, beta2, W1, b1, W2, b2):
    raise NotImplementedError("write your pallas kernel here")


# speedup vs baseline: 10.6862x; 10.6862x over previous
import functools

import jax
import jax.numpy as jnp
from jax import lax
from jax.experimental import pallas as pl
from jax.experimental.pallas import tpu as pltpu
from jax.experimental.pallas import tpu_sc as plsc

N = 100000
E = 6400000
FD = 11
ED = 4
G = 1000
C = 16
NC = 2
NS = 16
NW = NC * NS

EB = 128
NB = 8
CH = EB * NB
E_NCHUNK = E // CH
G_NB = 4
G_CH = EB * G_NB
G_NCHUNK = E // G_CH
NPAD = 102400
P_NCHUNK = NPAD // CH
NACC = 100352
GACC = 1024


def _chunk_counts(total):
    q, r = divmod(total, NW)
    def nk(w):
        return q + jnp.where(w < r, 1, 0).astype(jnp.int32)
    return nk

_mesh = plsc.VectorSubcoreMesh(core_axis_name="c", subcore_axis_name="s")


def _wid():
    return lax.axis_index("s") * NC + lax.axis_index("c")


def _gather_body(table, idxd, idxs, xd_out, xs_out,
                 table_sh, idxd_v, idxs_v, rowsd_v, rowss_v, semi, semg, semo):
    w = _wid()
    tid = lax.axis_index("s")
    rpt = NACC // NS
    nk = _chunk_counts(G_NCHUNK)(w)

    pltpu.sync_copy(table.at[pl.ds(tid * rpt, rpt)],
                    table_sh.at[pl.ds(tid * rpt, rpt)])
    plsc.subcore_barrier()

    def chunk(k, _):
        g = w + NW * k
        rb = g * G_NB
        eb = g * G_CH
        c1 = pltpu.async_copy(idxd.at[pl.ds(rb, G_NB)], idxd_v, semi)
        c2 = pltpu.async_copy(idxs.at[pl.ds(rb, G_NB)], idxs_v, semi)
        c1.wait()
        c2.wait()

        def fire(j, _):
            pltpu.async_copy(table_sh.at[idxd_v.at[j]],
                             rowsd_v.at[pl.ds(j * EB, EB)], semg)
            pltpu.async_copy(table_sh.at[idxs_v.at[j]],
                             rowss_v.at[pl.ds(j * EB, EB)], semg)
            return _
        lax.fori_loop(0, G_NB, fire, 0)

        def drain(j, _):
            pltpu.make_async_copy(table_sh.at[idxd_v.at[j]],
                                  rowsd_v.at[pl.ds(j * EB, EB)], semg).wait()
            pltpu.make_async_copy(table_sh.at[idxs_v.at[j]],
                                  rowss_v.at[pl.ds(j * EB, EB)], semg).wait()
            return _
        lax.fori_loop(0, G_NB, drain, 0)

        d1 = pltpu.async_copy(rowsd_v, xd_out.at[pl.ds(eb, G_CH)], semo)
        d2 = pltpu.async_copy(rowss_v, xs_out.at[pl.ds(eb, G_CH)], semo)
        d1.wait()
        d2.wait()
        return _

    lax.fori_loop(0, nk, chunk, 0)


@jax.jit
def _sc_gather(table, idxd2, idxs2):
    f = pl.kernel(
        _gather_body,
        out_type=(jax.ShapeDtypeStruct((E, C), jnp.float32),
                  jax.ShapeDtypeStruct((E, C), jnp.float32)),
        mesh=_mesh,
        scratch_types=[
            pltpu.VMEM_SHARED((NACC, C), jnp.float32),
            pltpu.VMEM((G_NB, EB), jnp.int32),
            pltpu.VMEM((G_NB, EB), jnp.int32),
            pltpu.VMEM((G_CH, C), jnp.float32),
            pltpu.VMEM((G_CH, C), jnp.float32),
            pltpu.SemaphoreType.DMA,
            pltpu.SemaphoreType.DMA,
            pltpu.SemaphoreType.DMA,
        ],
        compiler_params=pltpu.CompilerParams(use_tc_tiling_on_sc=False),
    )
    return f(table, idxd2, idxs2)


def _scatter_body(nout, nchunk, msg, idx2, zeros, out,
                  idx_v, msg_v, acc_sh, semi, semg):
    w = _wid()
    cid = lax.axis_index("c")
    tid = lax.axis_index("s")
    rpt = nout // NS
    nk = _chunk_counts(nchunk)(w)

    pltpu.sync_copy(zeros.at[pl.ds(tid * rpt, rpt)],
                    acc_sh.at[pl.ds(tid * rpt, rpt)])
    plsc.subcore_barrier()

    def chunk(k, _):
        g = w + NW * k
        c1 = pltpu.async_copy(idx2.at[pl.ds(g * NB, NB)], idx_v, semi)
        c2 = pltpu.async_copy(msg.at[pl.ds(g * CH, CH)], msg_v, semi)
        c1.wait()
        c2.wait()

        def fire(j, _):
            pltpu.async_copy(msg_v.at[pl.ds(j * EB, EB)],
                             acc_sh.at[idx_v.at[j]], semg, add=True)
            return _
        lax.fori_loop(0, NB, fire, 0)

        def drain(j, _):
            pltpu.make_async_copy(msg_v.at[pl.ds(j * EB, EB)],
                                  acc_sh.at[idx_v.at[j]], semg).wait()
            return _
        lax.fori_loop(0, NB, drain, 0)
        return _

    lax.fori_loop(0, nk, chunk, 0)
    plsc.subcore_barrier()
    pltpu.sync_copy(acc_sh.at[pl.ds(tid * rpt, rpt)],
                    out.at[cid, pl.ds(tid * rpt, rpt)])


def _make_scatter(nout, nchunk):
    body = functools.partial(_scatter_body, nout, nchunk)

    @jax.jit
    def run(msg, idx2, zeros):
        f = pl.kernel(
            body,
            out_type=jax.ShapeDtypeStruct((NC, nout, C), jnp.float32),
            mesh=_mesh,
            scratch_types=[
                pltpu.VMEM((NB, EB), jnp.int32),
                pltpu.VMEM((CH, C), jnp.float32),
                pltpu.VMEM_SHARED((nout, C), jnp.float32),
                pltpu.SemaphoreType.DMA,
                pltpu.SemaphoreType.DMA,
            ],
            compiler_params=pltpu.CompilerParams(use_tc_tiling_on_sc=False),
        )
        return f(msg, idx2, zeros)

    return run


_sc_scatter_edges = _make_scatter(NACC, E_NCHUNK)
_sc_scatter_pool = _make_scatter(GACC, P_NCHUNK)


ROWS = E // 8
MB = 1600
MGRID = ROWS // MB


def _msg_kernel(xd, xs, ea, wdf, wsf, wef, bf, wds, wss, wes, bs, out):
    zf = (jnp.dot(xd[...], wdf[...], preferred_element_type=jnp.float32)
          + jnp.dot(xs[...], wsf[...], preferred_element_type=jnp.float32)
          + jnp.dot(ea[...], wef[...], preferred_element_type=jnp.float32)
          + bf[...])
    zs = (jnp.dot(xd[...], wds[...], preferred_element_type=jnp.float32)
          + jnp.dot(xs[...], wss[...], preferred_element_type=jnp.float32)
          + jnp.dot(ea[...], wes[...], preferred_element_type=jnp.float32)
          + bs[...])
    out[...] = jax.nn.sigmoid(zf) * jax.nn.softplus(zs)


@jax.jit
def _tc_msg(xdp, xsp, eap, wdf, wsf, wef, bf, wds, wss, wes, bs):
    full = lambda s: pl.BlockSpec(s, lambda i: (0, 0))
    return pl.pallas_call(
        _msg_kernel,
        grid=(MGRID,),
        in_specs=[
            pl.BlockSpec((MB, 128), lambda i: (i, 0)),
            pl.BlockSpec((MB, 128), lambda i: (i, 0)),
            pl.BlockSpec((MB, 32), lambda i: (i, 0)),
            full((128, 128)), full((128, 128)), full((32, 128)), full((1, 128)),
            full((128, 128)), full((128, 128)), full((32, 128)), full((1, 128)),
        ],
        out_specs=pl.BlockSpec((MB, 128), lambda i: (i, 0)),
        out_shape=jax.ShapeDtypeStruct((ROWS, 128), jnp.float32),
    )(xdp, xsp, eap, wdf, wsf, wef, bf, wds, wss, wes, bs)


BNR = NPAD // 8
BNB = 1280
BNGRID = BNR // BNB


def _bn_stats_kernel(a0, a1, out):
    i = pl.program_id(0)
    agg = a0[...] + a1[...]
    s = jnp.sum(agg, axis=0, keepdims=True)
    s2 = jnp.sum(agg * agg, axis=0, keepdims=True)
    blk = jnp.concatenate([s, s2], axis=0)

    @pl.when(i == 0)
    def _():
        out[...] = blk

    @pl.when(i > 0)
    def _():
        out[...] = out[...] + blk


def _bn_norm_kernel(a0, a1, x, stats, fold, gamma, beta, out):
    st = jnp.dot(stats[...], fold[...], preferred_element_type=jnp.float32)
    mean = st[0:1, :] * (1.0 / N)
    var = st[1:2, :] * (1.0 / N) - mean * mean
    rstd = lax.rsqrt(var + 1e-5)
    agg = a0[...] + a1[...]
    h = (agg - mean) * rstd * gamma[...] + beta[...] + x[...]
    lanes = lax.broadcasted_iota(jnp.int32, h.shape, 1)
    out[...] = jnp.where(lanes % 16 == 11, 1.0, h)


@jax.jit
def _tc_bn(a0p, a1p, xp, fold, gamma, beta):
    full = lambda s: pl.BlockSpec(s, lambda i: (0, 0))
    stats = pl.pallas_call(
        _bn_stats_kernel,
        grid=(BNGRID,),
        in_specs=[pl.BlockSpec((BNB, 128), lambda i: (i, 0)),
                  pl.BlockSpec((BNB, 128), lambda i: (i, 0))],
        out_specs=pl.BlockSpec((2, 128), lambda i: (0, 0)),
        out_shape=jax.ShapeDtypeStruct((2, 128), jnp.float32),
    )(a0p, a1p)
    return pl.pallas_call(
        _bn_norm_kernel,
        grid=(BNGRID,),
        in_specs=[pl.BlockSpec((BNB, 128), lambda i: (i, 0)),
                  pl.BlockSpec((BNB, 128), lambda i: (i, 0)),
                  pl.BlockSpec((BNB, 128), lambda i: (i, 0)),
                  full((2, 128)), full((128, 128)),
                  full((1, 128)), full((1, 128))],
        out_specs=pl.BlockSpec((BNB, 128), lambda i: (i, 0)),
        out_shape=jax.ShapeDtypeStruct((BNR, 128), jnp.float32),
    )(a0p, a1p, xp, stats, fold, gamma, beta)


def _mlp_kernel(p0, p1, e11, w1, b1, w2, b2, out):
    ps = p0[...] + p1[...]
    cnt = jnp.clip(jnp.dot(ps, e11[...], preferred_element_type=jnp.float32),
                   1.0, None)
    pooled = ps / cnt
    h2 = jax.nn.softplus(
        jnp.dot(pooled, w1[...], preferred_element_type=jnp.float32) + b1[...])
    out[...] = jnp.dot(h2, w2[...], preferred_element_type=jnp.float32) + b2[...]


@jax.jit
def _tc_mlp(p0, p1, e11, w1, b1, w2, b2):
    full = lambda s: pl.BlockSpec(s, lambda: (0, 0))
    return pl.pallas_call(
        _mlp_kernel,
        in_specs=[full((GACC, C)), full((GACC, C)), full((C, C)),
                  full((C, 8)), full((1, 8)), full((8, 8)), full((1, 8))],
        out_specs=full((GACC, 8)),
        out_shape=jax.ShapeDtypeStruct((GACC, 8), jnp.float32),
    )(p0, p1, e11, w1, b1, w2, b2)


def _layer_weights(Wf, bf, Ws, bs):
    def bd(A, reps):
        return jnp.kron(jnp.eye(reps, dtype=jnp.float32), A)

    def padw(Wsub, rows):
        A = jnp.zeros((rows, C), jnp.float32)
        return A.at[:Wsub.shape[1], :FD].set(Wsub.T)

    wdf = bd(padw(Wf[:, :FD], C), 8)
    wsf = bd(padw(Wf[:, FD:2 * FD], C), 8)
    wef = bd(padw(Wf[:, 2 * FD:], ED), 8)
    wds = bd(padw(Ws[:, :FD], C), 8)
    wss = bd(padw(Ws[:, FD:2 * FD], C), 8)
    wes = bd(padw(Ws[:, 2 * FD:], ED), 8)
    bfp = jnp.tile(jnp.pad(bf, (0, C - FD))[None, :], (1, 8))
    bsp = jnp.tile(jnp.pad(bs, (0, C - FD))[None, :], (1, 8))
    return wdf, wsf, wef, bfp, wds, wss, wes, bsp


def _cgconv_layer(hp, idxd2, idxs2, dst2, eap, zeros_n, fold, gamma, beta, wts):
    wdf, wsf, wef, bfp, wds, wss, wes, bsp = wts
    xd, xs = _sc_gather(jnp.pad(hp, ((0, NACC - N), (0, 0))), idxd2, idxs2)
    msg = _tc_msg(xd.reshape(ROWS, 128), xs.reshape(ROWS, 128), eap,
                  wdf, wsf, wef, bfp, wds, wss, wes, bsp)
    acc = _sc_scatter_edges(msg.reshape(E, C), dst2, zeros_n)

    def pk(a):
        return jnp.pad(a.reshape(N // 8, 128), ((0, BNR - N // 8), (0, 0)))

    hpk = _tc_bn(pk(acc[0, :N]), pk(acc[1, :N]), pk(hp), fold, gamma, beta)
    return hpk


def kernel(x, edge_index, edge_attr, batch, Wf1, bf1, Ws1, bs1, gamma1, beta1,
           Wf2, bf2, Ws2, bs2, gamma2, beta2, W1, b1, W2, b2):
    xp = jnp.pad(x, ((0, 0), (0, C - FD)))
    src2 = edge_index[0].reshape(E // EB, EB)
    dst2 = edge_index[1].reshape(E // EB, EB)
    eap = edge_attr.reshape(ROWS, 8 * ED)
    zeros_n = jnp.zeros((NACC, C), jnp.float32)
    zeros_g = jnp.zeros((GACC, C), jnp.float32)
    fold = jnp.kron(jnp.ones((8, 1), jnp.float32), jnp.eye(C, dtype=jnp.float32))
    fold = fold.reshape(128, C)
    foldp = jnp.tile(fold, (1, 8))
    g1 = jnp.tile(jnp.pad(gamma1, (0, C - FD))[None, :], (1, 8))
    b1p = jnp.tile(jnp.pad(beta1, (0, C - FD))[None, :], (1, 8))
    g2 = jnp.tile(jnp.pad(gamma2, (0, C - FD))[None, :], (1, 8))
    b2p = jnp.tile(jnp.pad(beta2, (0, C - FD))[None, :], (1, 8))
    wts1 = _layer_weights(Wf1, bf1, Ws1, bs1)
    wts2 = _layer_weights(Wf2, bf2, Ws2, bs2)
    batch_pad = jnp.concatenate([batch, jnp.full((NPAD - N,), G, jnp.int32)])
    batch2 = batch_pad.reshape(NPAD // EB, EB)
    e11 = jnp.zeros((C, C), jnp.float32).at[FD, :].set(1.0)
    w1p = jnp.zeros((C, 8), jnp.float32).at[:FD, :5].set(W1.T)
    b1m = jnp.pad(b1, (0, 3))[None, :]
    w2p = jnp.zeros((8, 8), jnp.float32).at[:5, 0].set(W2[0])
    b2m = jnp.zeros((1, 8), jnp.float32).at[0, 0].set(b2[0])

    h1pk = _cgconv_layer(xp, dst2, src2, dst2, eap, zeros_n, foldp, g1, b1p, wts1)
    h1 = h1pk[:N // 8].reshape(N, C)
    h2pk = _cgconv_layer(h1, dst2, src2, dst2, eap, zeros_n, foldp, g2, b2p, wts2)

    pacc = _sc_scatter_pool(h2pk.reshape(NPAD, C), batch2, zeros_g)
    out = _tc_mlp(pacc[0], pacc[1], e11, w1p, b1m, w2p, b2m)
    return out[:G, :1]
